# Initial kernel scaffold; baseline (speedup 1.0000x reference)
import jax
import jax.numpy as jnp
from jax.experimental import pallas as pl


def kernel(user, item, cate, item_his, cate_his, user_table, item_table, cate_table):
    raise NotImplementedError("write your pallas kernel here")


# speedup vs baseline: 1.4059x; 1.4059x over previous
import functools

import jax
import jax.numpy as jnp
from jax import lax
from jax.experimental import pallas as pl
from jax.experimental.pallas import tpu as pltpu
from jax.experimental.pallas import tpu_sc as plsc

B = 4096
L = 200
D = 32
NC = 2
NS = 16
NW = NC * NS
BPW = B // NW
CB = 4
NCHUNK = BPW // CB
ROWS_PER_CHUNK = CB * 2 * L
GW = 64
NG = ROWS_PER_CHUNK // GW
IDX_ROWS_PER_W = BPW * 2 * L // GW


def _sc_body(user_i, item_i, cate_i, his_idx, user_table, item_table,
             cate_table, user_out, join_out, his_out, sum_out,
             idx_v, data_v, sum_v, sidx_v, rows_a, rows_b, join_v, sem):
    wid = lax.axis_index("s") * NC + lax.axis_index("c")
    base = wid * BPW

    pltpu.sync_copy(user_i.at[pl.ds(base, BPW)], sidx_v)
    pltpu.async_copy(user_table.at[sidx_v], rows_a, sem).wait()
    pltpu.sync_copy(rows_a, user_out.at[pl.ds(base, BPW)])

    pltpu.sync_copy(item_i.at[pl.ds(base, BPW)], sidx_v)
    pltpu.async_copy(item_table.at[sidx_v], rows_a, sem).wait()
    pltpu.sync_copy(cate_i.at[pl.ds(base, BPW)], sidx_v)
    pltpu.async_copy(cate_table.at[sidx_v], rows_b, sem).wait()

    def interleave(i, c):
        join_v[i, pl.ds(0, 16)] = rows_a[i, pl.ds(0, 16)]
        join_v[i, pl.ds(16, 16)] = rows_a[i, pl.ds(16, 16)]
        join_v[i, pl.ds(32, 16)] = rows_b[i, pl.ds(0, 16)]
        join_v[i, pl.ds(48, 16)] = rows_b[i, pl.ds(16, 16)]
        return c
    lax.fori_loop(0, BPW, interleave, 0)
    pltpu.sync_copy(join_v, join_out.at[pl.ds(base, BPW)])

    his_row0 = wid * BPW * 2 * L

    def chunk(g, c):
        pltpu.sync_copy(his_idx.at[wid * NCHUNK + g], idx_v)
        cps = [pltpu.async_copy(item_table.at[idx_v.at[j]],
                                data_v.at[pl.ds(j * GW, GW)], sem)
               for j in range(NG)]
        for cp in cps:
            cp.wait()
        for i in range(CB):
            rbase = i * 2 * L

            def red(l, accs):
                a0, a1, b0, b1 = accs
                r = rbase + 2 * l
                a0 = a0 + data_v[r, pl.ds(0, 16)]
                a1 = a1 + data_v[r, pl.ds(16, 16)]
                b0 = b0 + data_v[r + 1, pl.ds(0, 16)]
                b1 = b1 + data_v[r + 1, pl.ds(16, 16)]
                return (a0, a1, b0, b1)

            z = jnp.zeros((16,), jnp.float32)
            a0, a1, b0, b1 = lax.fori_loop(0, L, red, (z, z, z, z),
                                           unroll=4)
            row = g * CB + i
            sum_v[row, pl.ds(0, 16)] = a0
            sum_v[row, pl.ds(16, 16)] = a1
            sum_v[row, pl.ds(32, 16)] = b0
            sum_v[row, pl.ds(48, 16)] = b1
        pltpu.sync_copy(data_v,
                        his_out.at[pl.ds(his_row0 + g * ROWS_PER_CHUNK,
                                         ROWS_PER_CHUNK)])
        return c

    lax.fori_loop(0, NCHUNK, chunk, 0)
    pltpu.sync_copy(sum_v, sum_out.at[pl.ds(base, BPW)])


@functools.partial(jax.jit, static_argnums=())
def _sc_call(user_i, item_i, cate_i, his_idx, user_table, item_table,
             cate_table):
    mesh = plsc.VectorSubcoreMesh(core_axis_name="c", subcore_axis_name="s")
    f = pl.kernel(
        _sc_body,
        out_type=(
            jax.ShapeDtypeStruct((B, D), jnp.float32),
            jax.ShapeDtypeStruct((B, 2 * D), jnp.float32),
            jax.ShapeDtypeStruct((B * 2 * L, D), jnp.float32),
            jax.ShapeDtypeStruct((B, 2 * D), jnp.float32),
        ),
        mesh=mesh,
        scratch_types=[
            pltpu.VMEM((NG, GW), jnp.int32),
            pltpu.VMEM((ROWS_PER_CHUNK, D), jnp.float32),
            pltpu.VMEM((BPW, 2 * D), jnp.float32),
            pltpu.VMEM((BPW,), jnp.int32),
            pltpu.VMEM((BPW, D), jnp.float32),
            pltpu.VMEM((BPW, D), jnp.float32),
            pltpu.VMEM((BPW, 2 * D), jnp.float32),
            pltpu.SemaphoreType.DMA,
        ],
        compiler_params=pltpu.CompilerParams(use_tc_tiling_on_sc=False),
    )
    return f(user_i, item_i, cate_i, his_idx, user_table, item_table,
             cate_table)


def kernel(user, item, cate, item_his, cate_his, user_table, item_table,
           cate_table):
    user = user.astype(jnp.int32)
    item = item.astype(jnp.int32)
    cate = cate.astype(jnp.int32)
    his_idx = jnp.stack([item_his.astype(jnp.int32),
                         cate_his.astype(jnp.int32)], axis=-1)
    his_idx = his_idx.reshape(NW * NCHUNK, NG, GW)
    user_emb, join_emb, his_flat, his_sum = _sc_call(
        user, item, cate, his_idx, user_table, item_table, cate_table)
    return (user_emb, join_emb, his_flat.reshape(B, L, 2 * D), his_sum)


# speedup vs baseline: 1.4140x; 1.0057x over previous
import functools

import jax
import jax.numpy as jnp
from jax import lax
from jax.experimental import pallas as pl
from jax.experimental.pallas import tpu as pltpu
from jax.experimental.pallas import tpu_sc as plsc

B = 4096
L = 200
D = 32
NC = 2
NS = 16
NW = NC * NS
BPW = B // NW
CB = 4
NCHUNK = BPW // CB
ROWS_PER_CHUNK = CB * 2 * L
GW = 64
NG = ROWS_PER_CHUNK // GW


def _sc_body(user_i, item_i, cate_i, his_idx, user_table, item_table,
             cate_table, user_out, join_out, his_out, sum_out,
             idx0, idx1, data0, data1, sum_v, sidx, gsem0, gsem1, sem):
    wid = lax.axis_index("s") * NC + lax.axis_index("c")
    base = wid * BPW
    his_row0 = wid * BPW * 2 * L
    chunk0 = wid * NCHUNK

    pltpu.sync_copy(user_i.at[pl.ds(base, BPW)], sidx)
    pltpu.async_copy(user_table.at[sidx], data0.at[pl.ds(0, BPW)], sem).wait()
    pltpu.sync_copy(data0.at[pl.ds(0, BPW)], user_out.at[pl.ds(base, BPW)])

    pltpu.sync_copy(item_i.at[pl.ds(base, BPW)], sidx)
    pltpu.async_copy(item_table.at[sidx], data0.at[pl.ds(0, BPW)], sem).wait()
    pltpu.sync_copy(cate_i.at[pl.ds(base, BPW)], sidx)
    pltpu.async_copy(cate_table.at[sidx], data0.at[pl.ds(BPW, BPW)],
                     sem).wait()

    def interleave(i, c):
        sum_v[i, pl.ds(0, 16)] = data0[i, pl.ds(0, 16)]
        sum_v[i, pl.ds(16, 16)] = data0[i, pl.ds(16, 16)]
        sum_v[i, pl.ds(32, 16)] = data0[BPW + i, pl.ds(0, 16)]
        sum_v[i, pl.ds(48, 16)] = data0[BPW + i, pl.ds(16, 16)]
        return c
    lax.fori_loop(0, BPW, interleave, 0)
    pltpu.sync_copy(sum_v, join_out.at[pl.ds(base, BPW)])

    def fire(c, idxb, datab, semb):
        pltpu.sync_copy(his_idx.at[chunk0 + c], idxb)
        for j in range(NG):
            pltpu.async_copy(item_table.at[idxb.at[j]],
                             datab.at[pl.ds(j * GW, GW)], semb)

    def drain(idxb, datab, semb):
        for j in range(NG):
            pltpu.make_async_copy(item_table.at[idxb.at[j]],
                                  datab.at[pl.ds(j * GW, GW)], semb).wait()

    def process(c, datab):
        for i in range(CB):
            rbase = i * 2 * L

            def red(l, accs):
                a0, a1, b0, b1 = accs
                r = rbase + 2 * l
                a0 = a0 + datab[r, pl.ds(0, 16)]
                a1 = a1 + datab[r, pl.ds(16, 16)]
                b0 = b0 + datab[r + 1, pl.ds(0, 16)]
                b1 = b1 + datab[r + 1, pl.ds(16, 16)]
                return (a0, a1, b0, b1)

            z = jnp.zeros((16,), jnp.float32)
            a0, a1, b0, b1 = lax.fori_loop(0, L, red, (z, z, z, z),
                                           unroll=4)
            row = c * CB + i
            sum_v[row, pl.ds(0, 16)] = a0
            sum_v[row, pl.ds(16, 16)] = a1
            sum_v[row, pl.ds(32, 16)] = b0
            sum_v[row, pl.ds(48, 16)] = b1
        pltpu.sync_copy(datab,
                        his_out.at[pl.ds(his_row0 + c * ROWS_PER_CHUNK,
                                         ROWS_PER_CHUNK)])

    fire(0, idx0, data0, gsem0)

    def pipe(k, carry):
        c0 = 2 * k
        c1 = 2 * k + 1
        c2 = 2 * k + 2
        fire(c1, idx1, data1, gsem1)
        drain(idx0, data0, gsem0)
        process(c0, data0)

        @pl.when(c2 < NCHUNK)
        def _():
            fire(c2, idx0, data0, gsem0)

        drain(idx1, data1, gsem1)
        process(c1, data1)
        return carry

    lax.fori_loop(0, NCHUNK // 2, pipe, 0)
    pltpu.sync_copy(sum_v, sum_out.at[pl.ds(base, BPW)])


@functools.partial(jax.jit, static_argnums=())
def _sc_call(user_i, item_i, cate_i, his_idx, user_table, item_table,
             cate_table):
    mesh = plsc.VectorSubcoreMesh(core_axis_name="c", subcore_axis_name="s")
    f = pl.kernel(
        _sc_body,
        out_type=(
            jax.ShapeDtypeStruct((B, D), jnp.float32),
            jax.ShapeDtypeStruct((B, 2 * D), jnp.float32),
            jax.ShapeDtypeStruct((B * 2 * L, D), jnp.float32),
            jax.ShapeDtypeStruct((B, 2 * D), jnp.float32),
        ),
        mesh=mesh,
        scratch_types=[
            pltpu.VMEM((NG, GW), jnp.int32),
            pltpu.VMEM((NG, GW), jnp.int32),
            pltpu.VMEM((ROWS_PER_CHUNK, D), jnp.float32),
            pltpu.VMEM((ROWS_PER_CHUNK, D), jnp.float32),
            pltpu.VMEM((BPW, 2 * D), jnp.float32),
            pltpu.VMEM((BPW,), jnp.int32),
            pltpu.SemaphoreType.DMA,
            pltpu.SemaphoreType.DMA,
            pltpu.SemaphoreType.DMA,
        ],
        compiler_params=pltpu.CompilerParams(use_tc_tiling_on_sc=False),
    )
    return f(user_i, item_i, cate_i, his_idx, user_table, item_table,
             cate_table)


def kernel(user, item, cate, item_his, cate_his, user_table, item_table,
           cate_table):
    user = user.astype(jnp.int32)
    item = item.astype(jnp.int32)
    cate = cate.astype(jnp.int32)
    his_idx = jnp.stack([item_his.astype(jnp.int32),
                         cate_his.astype(jnp.int32)], axis=-1)
    his_idx = his_idx.reshape(NW * NCHUNK, NG, GW)
    user_emb, join_emb, his_flat, his_sum = _sc_call(
        user, item, cate, his_idx, user_table, item_table, cate_table)
    return (user_emb, join_emb, his_flat.reshape(B, L, 2 * D), his_sum)


# speedup vs baseline: 2.2771x; 1.6104x over previous
import functools

import jax
import jax.numpy as jnp
from jax import lax
from jax.experimental import pallas as pl
from jax.experimental.pallas import tpu as pltpu
from jax.experimental.pallas import tpu_sc as plsc

B = 4096
L = 200
D = 32
NC = 2
NS = 16
NW = NC * NS
BPW = B // NW
CB = 4
NCHUNK = BPW // CB
NPAIR = NCHUNK // 2
ROWS_PER_CHUNK = CB * 2 * L
GW = 64
NG = ROWS_PER_CHUNK // GW
NL16 = L // 16
LREM = L - 16 * NL16


def _sc_body(user_i, item_i, cate_i, item_hisT, cate_hisT, user_table,
             item_table, cate_table, user_out, join_out, his_out, sum_out,
             comb0, comb1, data0, data1, sum_v, sidx, itblk, ctblk,
             gsem0, gsem1, sem):
    wid = lax.axis_index("s") * NC + lax.axis_index("c")
    base = wid * BPW
    his_row0 = wid * BPW * 2 * L
    lanes = lax.iota(jnp.int32, 16)

    pltpu.sync_copy(user_i.at[pl.ds(base, BPW)], sidx)
    pltpu.async_copy(user_table.at[sidx], data0.at[pl.ds(0, BPW)], sem).wait()
    pltpu.sync_copy(data0.at[pl.ds(0, BPW)], user_out.at[pl.ds(base, BPW)])

    pltpu.sync_copy(item_i.at[pl.ds(base, BPW)], sidx)
    pltpu.async_copy(item_table.at[sidx], data0.at[pl.ds(0, BPW)], sem).wait()
    pltpu.sync_copy(cate_i.at[pl.ds(base, BPW)], sidx)
    pltpu.async_copy(cate_table.at[sidx], data0.at[pl.ds(BPW, BPW)],
                     sem).wait()

    def interleave(i, c):
        sum_v[i, pl.ds(0, 16)] = data0[i, pl.ds(0, 16)]
        sum_v[i, pl.ds(16, 16)] = data0[i, pl.ds(16, 16)]
        sum_v[i, pl.ds(32, 16)] = data0[BPW + i, pl.ds(0, 16)]
        sum_v[i, pl.ds(48, 16)] = data0[BPW + i, pl.ds(16, 16)]
        return c
    lax.fori_loop(0, BPW, interleave, 0)
    pltpu.sync_copy(sum_v, join_out.at[pl.ds(base, BPW)])

    def load_blocks(k):
        b0 = base + 8 * k
        pltpu.sync_copy(item_hisT.at[:, pl.ds(b0, 8)], itblk)
        pltpu.sync_copy(cate_hisT.at[:, pl.ds(b0, 8)], ctblk)

    def build_comb(parity, combb):
        for i in range(CB):
            bi = CB * parity + i
            cidx = jnp.full((16,), bi, jnp.int32)

            def lgroup(j, c):
                lv = 16 * j + lanes
                iv = plsc.load_gather(itblk, [lv, cidx])
                cv = plsc.load_gather(ctblk, [lv, cidx])
                p = (i * 2 * L) + 2 * lv
                plsc.store_scatter(
                    combb, [lax.shift_right_logical(p, 6),
                            lax.bitwise_and(p, 63)], iv)
                p1 = p + 1
                plsc.store_scatter(
                    combb, [lax.shift_right_logical(p1, 6),
                            lax.bitwise_and(p1, 63)], cv)
                return c

            lax.fori_loop(0, NL16, lgroup, 0)
            msk = lanes < LREM
            lv = 16 * NL16 + jnp.minimum(lanes, LREM - 1)
            iv = plsc.load_gather(itblk, [lv, cidx], mask=msk)
            cv = plsc.load_gather(ctblk, [lv, cidx], mask=msk)
            p = (i * 2 * L) + 2 * lv
            plsc.store_scatter(combb, [lax.shift_right_logical(p, 6),
                                       lax.bitwise_and(p, 63)], iv, mask=msk)
            p1 = p + 1
            plsc.store_scatter(combb, [lax.shift_right_logical(p1, 6),
                                       lax.bitwise_and(p1, 63)], cv, mask=msk)

    def fire(combb, datab, semb):
        for j in range(NG):
            pltpu.async_copy(item_table.at[combb.at[j]],
                             datab.at[pl.ds(j * GW, GW)], semb)

    def drain(combb, datab, semb):
        for j in range(NG):
            pltpu.make_async_copy(item_table.at[combb.at[j]],
                                  datab.at[pl.ds(j * GW, GW)], semb).wait()

    def process(c, datab):
        for i in range(CB):
            rbase = i * 2 * L

            def red(l, accs):
                a0, a1, b0, b1 = accs
                r = rbase + 2 * l
                a0 = a0 + datab[r, pl.ds(0, 16)]
                a1 = a1 + datab[r, pl.ds(16, 16)]
                b0 = b0 + datab[r + 1, pl.ds(0, 16)]
                b1 = b1 + datab[r + 1, pl.ds(16, 16)]
                return (a0, a1, b0, b1)

            z = jnp.zeros((16,), jnp.float32)
            a0, a1, b0, b1 = lax.fori_loop(0, L, red, (z, z, z, z),
                                           unroll=4)
            row = c * CB + i
            sum_v[row, pl.ds(0, 16)] = a0
            sum_v[row, pl.ds(16, 16)] = a1
            sum_v[row, pl.ds(32, 16)] = b0
            sum_v[row, pl.ds(48, 16)] = b1
        pltpu.sync_copy(datab,
                        his_out.at[pl.ds(his_row0 + c * ROWS_PER_CHUNK,
                                         ROWS_PER_CHUNK)])

    load_blocks(0)
    build_comb(0, comb0)
    fire(comb0, data0, gsem0)

    def pipe(k, carry):
        c0 = 2 * k
        c1 = 2 * k + 1
        c2 = 2 * k + 2
        build_comb(1, comb1)
        fire(comb1, data1, gsem1)

        @pl.when(k < NPAIR - 1)
        def _():
            load_blocks(k + 1)

        drain(comb0, data0, gsem0)
        process(c0, data0)

        @pl.when(c2 < NCHUNK)
        def _():
            build_comb(0, comb0)
            fire(comb0, data0, gsem0)

        drain(comb1, data1, gsem1)
        process(c1, data1)
        return carry

    lax.fori_loop(0, NPAIR, pipe, 0)
    pltpu.sync_copy(sum_v, sum_out.at[pl.ds(base, BPW)])


@functools.partial(jax.jit, static_argnums=())
def _sc_call(user_i, item_i, cate_i, item_hisT, cate_hisT, user_table,
             item_table, cate_table):
    mesh = plsc.VectorSubcoreMesh(core_axis_name="c", subcore_axis_name="s")
    f = pl.kernel(
        _sc_body,
        out_type=(
            jax.ShapeDtypeStruct((B, D), jnp.float32),
            jax.ShapeDtypeStruct((B, 2 * D), jnp.float32),
            jax.ShapeDtypeStruct((B * 2 * L, D), jnp.float32),
            jax.ShapeDtypeStruct((B, 2 * D), jnp.float32),
        ),
        mesh=mesh,
        scratch_types=[
            pltpu.VMEM((NG, GW), jnp.int32),
            pltpu.VMEM((NG, GW), jnp.int32),
            pltpu.VMEM((ROWS_PER_CHUNK, D), jnp.float32),
            pltpu.VMEM((ROWS_PER_CHUNK, D), jnp.float32),
            pltpu.VMEM((BPW, 2 * D), jnp.float32),
            pltpu.VMEM((BPW,), jnp.int32),
            pltpu.VMEM((L, 8), jnp.int32),
            pltpu.VMEM((L, 8), jnp.int32),
            pltpu.SemaphoreType.DMA,
            pltpu.SemaphoreType.DMA,
            pltpu.SemaphoreType.DMA,
        ],
        compiler_params=pltpu.CompilerParams(use_tc_tiling_on_sc=False,
                                             needs_layout_passes=False),
    )
    return f(user_i, item_i, cate_i, item_hisT, cate_hisT, user_table,
             item_table, cate_table)


def kernel(user, item, cate, item_his, cate_his, user_table, item_table,
           cate_table):
    user = user.astype(jnp.int32)
    item = item.astype(jnp.int32)
    cate = cate.astype(jnp.int32)
    item_hisT = item_his.astype(jnp.int32).T
    cate_hisT = cate_his.astype(jnp.int32).T
    user_emb, join_emb, his_flat, his_sum = _sc_call(
        user, item, cate, item_hisT, cate_hisT, user_table, item_table,
        cate_table)
    return (user_emb, join_emb, his_flat.reshape(B, L, 2 * D), his_sum)


# speedup vs baseline: 2.8824x; 1.2658x over previous
import functools

import jax
import jax.numpy as jnp
from jax import lax
from jax.experimental import pallas as pl
from jax.experimental.pallas import tpu as pltpu
from jax.experimental.pallas import tpu_sc as plsc

B = 4096
L = 200
D = 32
NC = 2
NS = 16
NW = NC * NS
BPW = B // NW
CB = 4
NCHUNK = BPW // CB
NPAIR = NCHUNK // 2
ROWS_PER_CHUNK = CB * 2 * L
GW = 64
NG = ROWS_PER_CHUNK // GW
NL16 = L // 16
LREM = L - 16 * NL16
RING = 4


def _stage1_body(user_i, item_i, cate_i, user_tT, item_tT, cate_tT,
                 user_out, join_out, ring, cate_v, stage_u, stage_j,
                 uidx, iidx, cidx, sem, csem):
    wid = lax.axis_index("s") * NC + lax.axis_index("c")
    base = wid * BPW
    lanes = lax.iota(jnp.int32, 16)

    pltpu.sync_copy(user_i.at[pl.ds(base, BPW)], uidx)
    pltpu.sync_copy(item_i.at[pl.ds(base, BPW)], iidx)
    pltpu.sync_copy(cate_i.at[pl.ds(base, BPW)], cidx)
    for k in range(8):
        pltpu.async_copy(cate_tT.at[:, pl.ds(128 * k, 128)], cate_v.at[k],
                         csem)
    for k in range(8):
        pltpu.make_async_copy(cate_tT.at[:, pl.ds(128 * k, 128)],
                              cate_v.at[k], csem).wait()

    def fire(su, si, u):
        r0 = 2 * (u % RING)
        pltpu.async_copy(
            user_tT.at[:, pl.ds(128 * lax.shift_right_logical(su, 7), 128)],
            ring.at[r0], sem)
        pltpu.async_copy(
            item_tT.at[:, pl.ds(128 * lax.shift_right_logical(si, 7), 128)],
            ring.at[r0 + 1], sem)

    def extract(buf_slot, col, dst, drow, dcol):
        sv = jnp.full((16,), buf_slot, jnp.int32)
        cv = jnp.full((16,), col, jnp.int32)
        lo = plsc.load_gather(ring, [sv, lanes, cv])
        hi = plsc.load_gather(ring, [sv, lanes + 16, cv])
        dst[drow, pl.ds(dcol, 16)] = lo
        dst[drow, pl.ds(dcol + 16, 16)] = hi

    def process(su, si, sc, u):
        r0 = 2 * (u % RING)
        pltpu.make_async_copy(
            user_tT.at[:, pl.ds(128 * lax.shift_right_logical(su, 7), 128)],
            ring.at[r0], sem).wait()
        pltpu.make_async_copy(
            item_tT.at[:, pl.ds(128 * lax.shift_right_logical(si, 7), 128)],
            ring.at[r0 + 1], sem).wait()
        extract(r0, lax.bitwise_and(su, 127), stage_u, u, 0)
        extract(r0 + 1, lax.bitwise_and(si, 127), stage_j, u, 0)
        ctv = jnp.full((16,), lax.shift_right_logical(sc, 7), jnp.int32)
        ccv = jnp.full((16,), lax.bitwise_and(sc, 127), jnp.int32)
        clo = plsc.load_gather(cate_v, [ctv, lanes, ccv])
        chi = plsc.load_gather(cate_v, [ctv, lanes + 16, ccv])
        stage_j[u, pl.ds(32, 16)] = clo
        stage_j[u, pl.ds(48, 16)] = chi

    def group(g, carry):
        u0 = 16 * g
        uv = uidx[pl.ds(u0, 16)]
        iv = iidx[pl.ds(u0, 16)]
        cv = cidx[pl.ds(u0, 16)]
        for k in range(RING):
            fire(uv[k], iv[k], u0 + k)
        for k in range(16):
            process(uv[k], iv[k], cv[k], u0 + k)
            if k + RING < 16:
                fire(uv[k + RING], iv[k + RING], u0 + k + RING)
        return carry

    lax.fori_loop(0, BPW // 16, group, 0)

    pltpu.sync_copy(stage_u, user_out.at[pl.ds(base, BPW)])
    pltpu.sync_copy(stage_j, join_out.at[pl.ds(base, BPW)])


def _his_body(item_hisT, cate_hisT, item_table, his_out, sum_out,
              comb0, comb1, data0, data1, sum_v, itblk, ctblk,
              gsem0, gsem1):
    wid = lax.axis_index("s") * NC + lax.axis_index("c")
    base = wid * BPW
    his_row0 = wid * BPW * 2 * L
    lanes = lax.iota(jnp.int32, 16)

    def load_blocks(k):
        b0 = base + 8 * k
        pltpu.sync_copy(item_hisT.at[:, pl.ds(b0, 8)], itblk)
        pltpu.sync_copy(cate_hisT.at[:, pl.ds(b0, 8)], ctblk)

    def build_comb(parity, combb):
        for i in range(CB):
            bi = CB * parity + i
            cidx = jnp.full((16,), bi, jnp.int32)

            def lgroup(j, c):
                lv = 16 * j + lanes
                iv = plsc.load_gather(itblk, [lv, cidx])
                cv = plsc.load_gather(ctblk, [lv, cidx])
                p = (i * 2 * L) + 2 * lv
                plsc.store_scatter(
                    combb, [lax.shift_right_logical(p, 6),
                            lax.bitwise_and(p, 63)], iv)
                p1 = p + 1
                plsc.store_scatter(
                    combb, [lax.shift_right_logical(p1, 6),
                            lax.bitwise_and(p1, 63)], cv)
                return c

            lax.fori_loop(0, NL16, lgroup, 0)
            msk = lanes < LREM
            lv = 16 * NL16 + jnp.minimum(lanes, LREM - 1)
            iv = plsc.load_gather(itblk, [lv, cidx], mask=msk)
            cv = plsc.load_gather(ctblk, [lv, cidx], mask=msk)
            p = (i * 2 * L) + 2 * lv
            plsc.store_scatter(combb, [lax.shift_right_logical(p, 6),
                                       lax.bitwise_and(p, 63)], iv, mask=msk)
            p1 = p + 1
            plsc.store_scatter(combb, [lax.shift_right_logical(p1, 6),
                                       lax.bitwise_and(p1, 63)], cv, mask=msk)

    def fire(combb, datab, semb):
        for j in range(NG):
            pltpu.async_copy(item_table.at[combb.at[j]],
                             datab.at[pl.ds(j * GW, GW)], semb)

    def drain(combb, datab, semb):
        for j in range(NG):
            pltpu.make_async_copy(item_table.at[combb.at[j]],
                                  datab.at[pl.ds(j * GW, GW)], semb).wait()

    def process(c, datab):
        for i in range(CB):
            rbase = i * 2 * L

            def red(l, accs):
                a0, a1, b0, b1 = accs
                r = rbase + 2 * l
                a0 = a0 + datab[r, pl.ds(0, 16)]
                a1 = a1 + datab[r, pl.ds(16, 16)]
                b0 = b0 + datab[r + 1, pl.ds(0, 16)]
                b1 = b1 + datab[r + 1, pl.ds(16, 16)]
                return (a0, a1, b0, b1)

            z = jnp.zeros((16,), jnp.float32)
            a0, a1, b0, b1 = lax.fori_loop(0, L, red, (z, z, z, z),
                                           unroll=4)
            row = c * CB + i
            sum_v[row, pl.ds(0, 16)] = a0
            sum_v[row, pl.ds(16, 16)] = a1
            sum_v[row, pl.ds(32, 16)] = b0
            sum_v[row, pl.ds(48, 16)] = b1
        pltpu.sync_copy(datab,
                        his_out.at[pl.ds(his_row0 + c * ROWS_PER_CHUNK,
                                         ROWS_PER_CHUNK)])

    load_blocks(0)
    build_comb(0, comb0)
    fire(comb0, data0, gsem0)

    def pipe(k, carry):
        c0 = 2 * k
        c1 = 2 * k + 1
        c2 = 2 * k + 2
        build_comb(1, comb1)
        fire(comb1, data1, gsem1)

        @pl.when(k < NPAIR - 1)
        def _():
            load_blocks(k + 1)

        drain(comb0, data0, gsem0)
        process(c0, data0)

        @pl.when(c2 < NCHUNK)
        def _():
            build_comb(0, comb0)
            fire(comb0, data0, gsem0)

        drain(comb1, data1, gsem1)
        process(c1, data1)
        return carry

    lax.fori_loop(0, NPAIR, pipe, 0)
    pltpu.sync_copy(sum_v, sum_out.at[pl.ds(base, BPW)])


@functools.partial(jax.jit, static_argnums=())
def _sc_call(user_i, item_i, cate_i, item_hisT, cate_hisT, user_tT,
             item_tT, cate_tT, item_table):
    mesh = plsc.VectorSubcoreMesh(core_axis_name="c", subcore_axis_name="s")
    stage1 = pl.kernel(
        _stage1_body,
        out_type=(
            jax.ShapeDtypeStruct((B, D), jnp.float32),
            jax.ShapeDtypeStruct((B, 2 * D), jnp.float32),
        ),
        mesh=mesh,
        scratch_types=[
            pltpu.VMEM((2 * RING, D, 128), jnp.float32),
            pltpu.VMEM((8, D, 128), jnp.float32),
            pltpu.VMEM((BPW, D), jnp.float32),
            pltpu.VMEM((BPW, 2 * D), jnp.float32),
            pltpu.VMEM((BPW,), jnp.int32),
            pltpu.VMEM((BPW,), jnp.int32),
            pltpu.VMEM((BPW,), jnp.int32),
            pltpu.SemaphoreType.DMA,
            pltpu.SemaphoreType.DMA,
        ],
        compiler_params=pltpu.CompilerParams(needs_layout_passes=False),
    )
    user_emb, join_emb = stage1(user_i, item_i, cate_i, user_tT, item_tT,
                                cate_tT)

    his = pl.kernel(
        _his_body,
        out_type=(
            jax.ShapeDtypeStruct((B * 2 * L, D), jnp.float32),
            jax.ShapeDtypeStruct((B, 2 * D), jnp.float32),
        ),
        mesh=mesh,
        scratch_types=[
            pltpu.VMEM((NG, GW), jnp.int32),
            pltpu.VMEM((NG, GW), jnp.int32),
            pltpu.VMEM((ROWS_PER_CHUNK, D), jnp.float32),
            pltpu.VMEM((ROWS_PER_CHUNK, D), jnp.float32),
            pltpu.VMEM((BPW, 2 * D), jnp.float32),
            pltpu.VMEM((L, 8), jnp.int32),
            pltpu.VMEM((L, 8), jnp.int32),
            pltpu.SemaphoreType.DMA,
            pltpu.SemaphoreType.DMA,
        ],
        compiler_params=pltpu.CompilerParams(use_tc_tiling_on_sc=False,
                                             needs_layout_passes=False),
    )
    his_flat, his_sum = his(item_hisT, cate_hisT, item_table)
    return user_emb, join_emb, his_flat, his_sum


def kernel(user, item, cate, item_his, cate_his, user_table, item_table,
           cate_table):
    user = user.astype(jnp.int32)
    item = item.astype(jnp.int32)
    cate = cate.astype(jnp.int32)
    item_hisT = item_his.astype(jnp.int32).T
    cate_hisT = cate_his.astype(jnp.int32).T
    cate_pad = jnp.pad(cate_table, ((0, 1024 - cate_table.shape[0]),
                                    (0, 0)))
    user_emb, join_emb, his_flat, his_sum = _sc_call(
        user, item, cate, item_hisT, cate_hisT, user_table.T, item_table.T,
        cate_pad.T, item_table)
    return (user_emb, join_emb, his_flat.reshape(B, L, 2 * D), his_sum)
